# baseline (speedup 1.0000x reference)
import functools

import jax
import jax.numpy as jnp
from jax import lax
from jax.experimental import pallas as pl
from jax.experimental.pallas import tpu as pltpu
from jax.experimental.pallas import tpu_sc as plsc

N = 10000
E = 320000
D = 128
DH = D // 2
ACT = 10
H = 32

NPAD = 10240
NC = 2
NS = 16
NW = NC * NS
KW = 80
CH = (E // NW) // KW
KW2 = 128
CH2 = 158
ROWS_PER_TILE = NPAD // NS
DEG_PER_TILE = NPAD // NS


def _sc_mesh():
    return plsc.VectorSubcoreMesh(core_axis_name="c", subcore_axis_name="s")


def _sc_degree(dst3, zeros_deg, ones_kw):
    @functools.partial(
        pl.kernel,
        mesh=_sc_mesh(),
        out_type=jax.ShapeDtypeStruct((NC * NPAD,), jnp.float32),
        scratch_types=[
            pltpu.VMEM((CH, KW), jnp.int32),
            pltpu.VMEM((KW,), jnp.float32),
            pltpu.VMEM_SHARED((NPAD,), jnp.float32),
        ],
    )
    def k(dst_hbm, z_hbm, ones_hbm, out_hbm, dbuf, ones_v, deg_sp):
        c = lax.axis_index("c")
        s = lax.axis_index("s")
        w = c * NS + s
        pltpu.sync_copy(z_hbm, deg_sp.at[pl.ds(s * DEG_PER_TILE, DEG_PER_TILE)])
        pltpu.sync_copy(ones_hbm, ones_v)
        pltpu.sync_copy(dst_hbm.at[w], dbuf)
        plsc.subcore_barrier()

        def body(j, carry):
            pltpu.sync_copy(ones_v, deg_sp.at[dbuf.at[j]], add=True)
            return carry

        lax.fori_loop(0, CH, body, 0)
        plsc.subcore_barrier()
        pltpu.sync_copy(
            deg_sp.at[pl.ds(s * DEG_PER_TILE, DEG_PER_TILE)],
            out_hbm.at[pl.ds(c * NPAD + s * DEG_PER_TILE, DEG_PER_TILE)],
        )

    return k(dst3, zeros_deg, ones_kw)


def _sc_aggregate(y2, src2, dst2, zeros_rows):
    @functools.partial(
        pl.kernel,
        mesh=_sc_mesh(),
        compiler_params=pltpu.CompilerParams(use_tc_tiling_on_sc=False),
        out_type=jax.ShapeDtypeStruct((NC, NPAD, DH), jnp.float32),
        scratch_types=[
            pltpu.VMEM((CH2, KW2), jnp.int32),
            pltpu.VMEM((CH2, KW2), jnp.int32),
            pltpu.VMEM((KW2, DH), jnp.float32),
            pltpu.VMEM((KW2, DH), jnp.float32),
            pltpu.SemaphoreType.DMA,
            pltpu.SemaphoreType.DMA,
            pltpu.SemaphoreType.DMA,
            pltpu.SemaphoreType.DMA,
            pltpu.VMEM_SHARED((NPAD, DH), jnp.float32),
        ],
    )
    def k(y_hbm, src_hbm, dst_hbm, z_hbm, out_hbm, sbuf, dbuf, rows0, rows1,
          sem0, sem1, sems0, sems1, agg_sp):
        c = lax.axis_index("c")
        s = lax.axis_index("s")
        pltpu.sync_copy(z_hbm, agg_sp.at[pl.ds(s * ROWS_PER_TILE, ROWS_PER_TILE)])
        pltpu.sync_copy(src_hbm.at[s], sbuf)
        pltpu.sync_copy(dst_hbm.at[s], dbuf)
        def shift(i, carry):
            r = i // (KW2 // 16)
            col = (i % (KW2 // 16)) * 16
            v = sbuf[r, pl.ds(col, 16)]
            sbuf[r, pl.ds(col, 16)] = v + v + c
            return carry

        lax.fori_loop(0, CH2 * (KW2 // 16), shift, 0)
        plsc.subcore_barrier()

        pltpu.async_copy(y_hbm.at[sbuf.at[0]], rows0, sem0)
        pltpu.async_copy(y_hbm.at[sbuf.at[1]], rows1, sem1)

        def body(g, carry):
            j0 = 2 * g
            pltpu.make_async_copy(y_hbm.at[sbuf.at[j0]], rows0, sem0).wait()
            pltpu.sync_copy(rows0, agg_sp.at[dbuf.at[j0]], add=True)
            pltpu.async_copy(y_hbm.at[sbuf.at[j0 + 2]], rows0, sem0)
            pltpu.make_async_copy(y_hbm.at[sbuf.at[j0 + 1]], rows1, sem1).wait()
            pltpu.sync_copy(rows1, agg_sp.at[dbuf.at[j0 + 1]], add=True)
            pltpu.async_copy(y_hbm.at[sbuf.at[j0 + 3]], rows1, sem1)
            return carry

        lax.fori_loop(0, CH2 // 2 - 1, body, 0)
        pltpu.make_async_copy(y_hbm.at[sbuf.at[CH2 - 2]], rows0, sem0).wait()
        pltpu.sync_copy(rows0, agg_sp.at[dbuf.at[CH2 - 2]], add=True)
        pltpu.make_async_copy(y_hbm.at[sbuf.at[CH2 - 1]], rows1, sem1).wait()
        pltpu.sync_copy(rows1, agg_sp.at[dbuf.at[CH2 - 1]], add=True)
        plsc.subcore_barrier()
        pltpu.sync_copy(
            agg_sp.at[pl.ds(s * ROWS_PER_TILE, ROWS_PER_TILE)],
            out_hbm.at[c, pl.ds(s * ROWS_PER_TILE, ROWS_PER_TILE)],
        )

    return k(y2, src2, dst2, zeros_rows)


def _tc_scale_matmul(state, Wc, degp):
    BM = 1000

    def body(x_ref, w_ref, d_ref, o_ref, dv_ref):
        deg = d_ref[:, 0:1] + d_ref[:, 1:2] + 1.0
        dv = lax.rsqrt(deg)
        dv_ref[...] = dv
        o_ref[...] = (
            jnp.dot(x_ref[...], w_ref[...], preferred_element_type=jnp.float32)
            * dv
        )

    return pl.pallas_call(
        body,
        grid=(N // BM,),
        in_specs=[
            pl.BlockSpec((BM, D), lambda i: (i, 0)),
            pl.BlockSpec((D, D), lambda i: (0, 0)),
            pl.BlockSpec((BM, NC), lambda i: (i, 0)),
        ],
        out_specs=[
            pl.BlockSpec((BM, D), lambda i: (i, 0)),
            pl.BlockSpec((BM, 1), lambda i: (i, 0)),
        ],
        out_shape=[
            jax.ShapeDtypeStruct((N, D), jnp.float32),
            jax.ShapeDtypeStruct((N, 1), jnp.float32),
        ],
    )(state, Wc, degp)


def _tc_post(aggp, y2, dinv_col, state, af, Amat, bc2, W1a, w1b, b12, W2, b22, W3, b32):
    BM = 2000
    BG = BM // ACT

    def body(agg_ref, y_ref, dv_ref, x_ref, a_ref, A_ref, bc_ref,
             W1_ref, w1b_ref, b1_ref, W2_ref, b2_ref, W3_ref, b3_ref, o_ref):
        aggsum = jnp.concatenate([agg_ref[0], agg_ref[1]], axis=-1)
        conv = jnp.maximum(dv_ref[...] * (aggsum + y_ref[...]) + bc_ref[...], 0.0)
        x = conv + x_ref[...]
        h1 = jnp.maximum(
            jnp.dot(x, W1_ref[...], preferred_element_type=jnp.float32)
            + a_ref[...] * w1b_ref[...]
            + b1_ref[...],
            0.0,
        )
        h2 = jnp.maximum(
            jnp.dot(h1, W2_ref[...], preferred_element_type=jnp.float32)
            + b2_ref[...],
            0.0,
        )
        hs = jnp.dot(A_ref[...], h2, preferred_element_type=jnp.float32)
        o_ref[...] = (
            jnp.dot(hs, W3_ref[...], preferred_element_type=jnp.float32)
            + b3_ref[...]
        )

    return pl.pallas_call(
        body,
        grid=(N // BM,),
        in_specs=[
            pl.BlockSpec((NC, BM, DH), lambda i: (0, i, 0)),
            pl.BlockSpec((BM, D), lambda i: (i, 0)),
            pl.BlockSpec((BM, 1), lambda i: (i, 0)),
            pl.BlockSpec((BM, D), lambda i: (i, 0)),
            pl.BlockSpec((BM, 1), lambda i: (i, 0)),
            pl.BlockSpec((BG, BM), lambda i: (0, 0)),
            pl.BlockSpec((1, D), lambda i: (0, 0)),
            pl.BlockSpec((D, H), lambda i: (0, 0)),
            pl.BlockSpec((1, H), lambda i: (0, 0)),
            pl.BlockSpec((1, H), lambda i: (0, 0)),
            pl.BlockSpec((H, H), lambda i: (0, 0)),
            pl.BlockSpec((1, H), lambda i: (0, 0)),
            pl.BlockSpec((H, 1), lambda i: (0, 0)),
            pl.BlockSpec((1, 1), lambda i: (0, 0)),
        ],
        out_specs=pl.BlockSpec((BG, 1), lambda i: (i, 0)),
        out_shape=jax.ShapeDtypeStruct((N // ACT, 1), jnp.float32),
    )(aggp, y2, dinv_col, state, af, Amat, bc2, W1a, w1b, b12, W2, b22, W3, b32)


def kernel(state, edge_index, action, Wc, bc, W1, b1, W2, b2, W3, b3):
    pad = CH2 * KW2 - E // NS
    src_pad = jnp.zeros((NS, pad), jnp.int32)
    dst_pad = jnp.broadcast_to(
        N + jnp.arange(pad, dtype=jnp.int32), (NS, pad))
    src2 = jnp.concatenate(
        [edge_index[0].reshape(NS, E // NS), src_pad], axis=1
    ).reshape(NS, CH2, KW2)
    dst2 = jnp.concatenate(
        [edge_index[1].reshape(NS, E // NS), dst_pad], axis=1
    ).reshape(NS, CH2, KW2)
    dst3 = edge_index[1].reshape(NW, CH, KW)
    af = action.reshape(N, 1)
    zeros_deg = jnp.zeros((DEG_PER_TILE,), jnp.float32)
    ones_kw = jnp.ones((KW,), jnp.float32)
    zeros_rows = jnp.zeros((ROWS_PER_TILE, DH), jnp.float32)
    BM = 2000
    BG = BM // ACT
    Amat = (jnp.arange(BM, dtype=jnp.int32) // ACT
            == jnp.arange(BG, dtype=jnp.int32)[:, None]).astype(jnp.float32)

    degp = _sc_degree(dst3, zeros_deg, ones_kw).reshape(NC, NPAD)
    degp_t = degp.T[:N]
    y, dinv_col = _tc_scale_matmul(state, Wc, degp_t)
    y2flat = y.reshape(NC * N, DH)
    aggp = _sc_aggregate(y2flat, src2, dst2, zeros_rows)
    out2 = _tc_post(
        aggp, y, dinv_col, state, af, Amat,
        bc.reshape(1, D), W1[:D], W1[D:], b1.reshape(1, H),
        W2, b2.reshape(1, H), W3, b3.reshape(1, 1),
    )
    return out2.reshape(N // ACT)


# speedup vs baseline: 1.0975x; 1.0975x over previous
import functools

import jax
import jax.numpy as jnp
from jax import lax
from jax.experimental import pallas as pl
from jax.experimental.pallas import tpu as pltpu
from jax.experimental.pallas import tpu_sc as plsc

N = 10000
E = 320000
D = 128
DH = D // 2
ACT = 10
H = 32

NPAD = 10240
NC = 2
NS = 16
NW = NC * NS
KW = 80
CH = (E // NW) // KW
KW2 = 64
CH2 = 314
ROWS_PER_TILE = NPAD // NS
DEG_PER_TILE = NPAD // NS


def _sc_mesh():
    return plsc.VectorSubcoreMesh(core_axis_name="c", subcore_axis_name="s")


def _sc_degree(dst3, zeros_deg, ones_kw):
    @functools.partial(
        pl.kernel,
        mesh=_sc_mesh(),
        out_type=jax.ShapeDtypeStruct((NC * NPAD,), jnp.float32),
        scratch_types=[
            pltpu.VMEM((CH, KW), jnp.int32),
            pltpu.VMEM((KW,), jnp.float32),
            pltpu.VMEM_SHARED((NPAD,), jnp.float32),
        ],
    )
    def k(dst_hbm, z_hbm, ones_hbm, out_hbm, dbuf, ones_v, deg_sp):
        c = lax.axis_index("c")
        s = lax.axis_index("s")
        w = c * NS + s
        pltpu.sync_copy(z_hbm, deg_sp.at[pl.ds(s * DEG_PER_TILE, DEG_PER_TILE)])
        pltpu.sync_copy(ones_hbm, ones_v)
        pltpu.sync_copy(dst_hbm.at[w], dbuf)
        plsc.subcore_barrier()

        def body(j, carry):
            pltpu.sync_copy(ones_v, deg_sp.at[dbuf.at[j]], add=True)
            return carry

        lax.fori_loop(0, CH, body, 0)
        plsc.subcore_barrier()
        pltpu.sync_copy(
            deg_sp.at[pl.ds(s * DEG_PER_TILE, DEG_PER_TILE)],
            out_hbm.at[pl.ds(c * NPAD + s * DEG_PER_TILE, DEG_PER_TILE)],
        )

    return k(dst3, zeros_deg, ones_kw)


def _sc_aggregate(y2, src2, dst2, zeros_rows):
    @functools.partial(
        pl.kernel,
        mesh=_sc_mesh(),
        compiler_params=pltpu.CompilerParams(use_tc_tiling_on_sc=False),
        out_type=jax.ShapeDtypeStruct((NC, NPAD, DH), jnp.float32),
        scratch_types=[
            pltpu.VMEM((CH2, KW2), jnp.int32),
            pltpu.VMEM((CH2, KW2), jnp.int32),
            pltpu.VMEM((KW2, DH), jnp.float32),
            pltpu.VMEM((KW2, DH), jnp.float32),
            pltpu.SemaphoreType.DMA,
            pltpu.SemaphoreType.DMA,
            pltpu.SemaphoreType.DMA,
            pltpu.SemaphoreType.DMA,
            pltpu.VMEM_SHARED((NPAD, DH), jnp.float32),
        ],
    )
    def k(y_hbm, src_hbm, dst_hbm, z_hbm, out_hbm, sbuf, dbuf, rows0, rows1,
          sem0, sem1, sems0, sems1, agg_sp):
        c = lax.axis_index("c")
        s = lax.axis_index("s")
        pltpu.sync_copy(z_hbm, agg_sp.at[pl.ds(s * ROWS_PER_TILE, ROWS_PER_TILE)])
        pltpu.sync_copy(src_hbm.at[s], sbuf)
        pltpu.sync_copy(dst_hbm.at[s], dbuf)
        def shift(i, carry):
            r = i // (KW2 // 16)
            col = (i % (KW2 // 16)) * 16
            v = sbuf[r, pl.ds(col, 16)]
            sbuf[r, pl.ds(col, 16)] = v + v + c
            return carry

        lax.fori_loop(0, CH2 * (KW2 // 16), shift, 0)
        plsc.subcore_barrier()

        pltpu.async_copy(y_hbm.at[sbuf.at[0]], rows0, sem0)
        pltpu.async_copy(y_hbm.at[sbuf.at[1]], rows1, sem1)

        def body(g, carry):
            j0 = 2 * g
            pltpu.make_async_copy(y_hbm.at[sbuf.at[j0]], rows0, sem0).wait()
            pltpu.sync_copy(rows0, agg_sp.at[dbuf.at[j0]], add=True)
            pltpu.async_copy(y_hbm.at[sbuf.at[j0 + 2]], rows0, sem0)
            pltpu.make_async_copy(y_hbm.at[sbuf.at[j0 + 1]], rows1, sem1).wait()
            pltpu.sync_copy(rows1, agg_sp.at[dbuf.at[j0 + 1]], add=True)
            pltpu.async_copy(y_hbm.at[sbuf.at[j0 + 3]], rows1, sem1)
            return carry

        lax.fori_loop(0, CH2 // 2 - 1, body, 0)
        pltpu.make_async_copy(y_hbm.at[sbuf.at[CH2 - 2]], rows0, sem0).wait()
        pltpu.sync_copy(rows0, agg_sp.at[dbuf.at[CH2 - 2]], add=True)
        pltpu.make_async_copy(y_hbm.at[sbuf.at[CH2 - 1]], rows1, sem1).wait()
        pltpu.sync_copy(rows1, agg_sp.at[dbuf.at[CH2 - 1]], add=True)
        plsc.subcore_barrier()
        pltpu.sync_copy(
            agg_sp.at[pl.ds(s * ROWS_PER_TILE, ROWS_PER_TILE)],
            out_hbm.at[c, pl.ds(s * ROWS_PER_TILE, ROWS_PER_TILE)],
        )

    return k(y2, src2, dst2, zeros_rows)


def _tc_scale_matmul(state, Wc, degp):
    BM = 1000

    def body(x_ref, w_ref, d_ref, o_ref, dv_ref):
        deg = d_ref[:, 0:1] + d_ref[:, 1:2] + 1.0
        dv = lax.rsqrt(deg)
        dv_ref[...] = dv
        o_ref[...] = (
            jnp.dot(x_ref[...], w_ref[...], preferred_element_type=jnp.float32)
            * dv
        )

    return pl.pallas_call(
        body,
        grid=(N // BM,),
        in_specs=[
            pl.BlockSpec((BM, D), lambda i: (i, 0)),
            pl.BlockSpec((D, D), lambda i: (0, 0)),
            pl.BlockSpec((BM, NC), lambda i: (i, 0)),
        ],
        out_specs=[
            pl.BlockSpec((BM, D), lambda i: (i, 0)),
            pl.BlockSpec((BM, 1), lambda i: (i, 0)),
        ],
        out_shape=[
            jax.ShapeDtypeStruct((N, D), jnp.float32),
            jax.ShapeDtypeStruct((N, 1), jnp.float32),
        ],
    )(state, Wc, degp)


def _tc_post(aggp, y2, dinv_col, state, af, Amat, bc2, W1a, w1b, b12, W2, b22, W3, b32):
    BM = 2000
    BG = BM // ACT

    def body(agg_ref, y_ref, dv_ref, x_ref, a_ref, A_ref, bc_ref,
             W1_ref, w1b_ref, b1_ref, W2_ref, b2_ref, W3_ref, b3_ref, o_ref):
        aggsum = jnp.concatenate([agg_ref[0], agg_ref[1]], axis=-1)
        conv = jnp.maximum(dv_ref[...] * (aggsum + y_ref[...]) + bc_ref[...], 0.0)
        x = conv + x_ref[...]
        h1 = jnp.maximum(
            jnp.dot(x, W1_ref[...], preferred_element_type=jnp.float32)
            + a_ref[...] * w1b_ref[...]
            + b1_ref[...],
            0.0,
        )
        h2 = jnp.maximum(
            jnp.dot(h1, W2_ref[...], preferred_element_type=jnp.float32)
            + b2_ref[...],
            0.0,
        )
        hs = jnp.dot(A_ref[...], h2, preferred_element_type=jnp.float32)
        o_ref[...] = (
            jnp.dot(hs, W3_ref[...], preferred_element_type=jnp.float32)
            + b3_ref[...]
        )

    return pl.pallas_call(
        body,
        grid=(N // BM,),
        in_specs=[
            pl.BlockSpec((NC, BM, DH), lambda i: (0, i, 0)),
            pl.BlockSpec((BM, D), lambda i: (i, 0)),
            pl.BlockSpec((BM, 1), lambda i: (i, 0)),
            pl.BlockSpec((BM, D), lambda i: (i, 0)),
            pl.BlockSpec((BM, 1), lambda i: (i, 0)),
            pl.BlockSpec((BG, BM), lambda i: (0, 0)),
            pl.BlockSpec((1, D), lambda i: (0, 0)),
            pl.BlockSpec((D, H), lambda i: (0, 0)),
            pl.BlockSpec((1, H), lambda i: (0, 0)),
            pl.BlockSpec((1, H), lambda i: (0, 0)),
            pl.BlockSpec((H, H), lambda i: (0, 0)),
            pl.BlockSpec((1, H), lambda i: (0, 0)),
            pl.BlockSpec((H, 1), lambda i: (0, 0)),
            pl.BlockSpec((1, 1), lambda i: (0, 0)),
        ],
        out_specs=pl.BlockSpec((BG, 1), lambda i: (i, 0)),
        out_shape=jax.ShapeDtypeStruct((N // ACT, 1), jnp.float32),
    )(aggp, y2, dinv_col, state, af, Amat, bc2, W1a, w1b, b12, W2, b22, W3, b32)


def kernel(state, edge_index, action, Wc, bc, W1, b1, W2, b2, W3, b3):
    pad = CH2 * KW2 - E // NS
    src_pad = jnp.zeros((NS, pad), jnp.int32)
    dst_pad = jnp.broadcast_to(
        N + jnp.arange(pad, dtype=jnp.int32), (NS, pad))
    src2 = jnp.concatenate(
        [edge_index[0].reshape(NS, E // NS), src_pad], axis=1
    ).reshape(NS, CH2, KW2)
    dst2 = jnp.concatenate(
        [edge_index[1].reshape(NS, E // NS), dst_pad], axis=1
    ).reshape(NS, CH2, KW2)
    dst3 = edge_index[1].reshape(NW, CH, KW)
    af = action.reshape(N, 1)
    zeros_deg = jnp.zeros((DEG_PER_TILE,), jnp.float32)
    ones_kw = jnp.ones((KW,), jnp.float32)
    zeros_rows = jnp.zeros((ROWS_PER_TILE, DH), jnp.float32)
    BM = 2000
    BG = BM // ACT
    Amat = (jnp.arange(BM, dtype=jnp.int32) // ACT
            == jnp.arange(BG, dtype=jnp.int32)[:, None]).astype(jnp.float32)

    degp = _sc_degree(dst3, zeros_deg, ones_kw).reshape(NC, NPAD)
    degp_t = degp.T[:N]
    y, dinv_col = _tc_scale_matmul(state, Wc, degp_t)
    y2flat = y.reshape(NC * N, DH)
    aggp = _sc_aggregate(y2flat, src2, dst2, zeros_rows)
    out2 = _tc_post(
        aggp, y, dinv_col, state, af, Amat,
        bc.reshape(1, D), W1[:D], W1[D:], b1.reshape(1, H),
        W2, b2.reshape(1, H), W3, b3.reshape(1, 1),
    )
    return out2.reshape(N // ACT)


# speedup vs baseline: 1.4147x; 1.2890x over previous
import functools

import jax
import jax.numpy as jnp
from jax import lax
from jax.experimental import pallas as pl
from jax.experimental.pallas import tpu as pltpu
from jax.experimental.pallas import tpu_sc as plsc

N = 10000
E = 320000
D = 128
DH = D // 2
ACT = 10
H = 32

NPAD = 10240
NC = 2
NS = 16
NW = NC * NS
KW = 80
CH = (E // NW) // KW
CH2 = (E // NS) // KW
ROWS_PER_TILE = NPAD // NS
DEG_PER_TILE = NPAD // NS


def _sc_mesh():
    return plsc.VectorSubcoreMesh(core_axis_name="c", subcore_axis_name="s")


def _sc_degree(dst3, zeros_deg, ones_kw):
    @functools.partial(
        pl.kernel,
        mesh=_sc_mesh(),
        out_type=jax.ShapeDtypeStruct((NC * NPAD,), jnp.float32),
        scratch_types=[
            pltpu.VMEM((CH, KW), jnp.int32),
            pltpu.VMEM((KW,), jnp.float32),
            pltpu.VMEM_SHARED((NPAD,), jnp.float32),
        ],
    )
    def k(dst_hbm, z_hbm, ones_hbm, out_hbm, dbuf, ones_v, deg_sp):
        c = lax.axis_index("c")
        s = lax.axis_index("s")
        w = c * NS + s
        pltpu.sync_copy(z_hbm, deg_sp.at[pl.ds(s * DEG_PER_TILE, DEG_PER_TILE)])
        pltpu.sync_copy(ones_hbm, ones_v)
        pltpu.sync_copy(dst_hbm.at[w], dbuf)
        plsc.subcore_barrier()

        def body(j, carry):
            pltpu.sync_copy(ones_v, deg_sp.at[dbuf.at[j]], add=True)
            return carry

        lax.fori_loop(0, CH, body, 0)
        plsc.subcore_barrier()
        pltpu.sync_copy(
            deg_sp.at[pl.ds(s * DEG_PER_TILE, DEG_PER_TILE)],
            out_hbm.at[pl.ds(c * NPAD + s * DEG_PER_TILE, DEG_PER_TILE)],
        )

    return k(dst3, zeros_deg, ones_kw)


def _sc_aggregate(y2, src2, dst2, zeros_rows):
    @functools.partial(
        pl.kernel,
        mesh=_sc_mesh(),
        compiler_params=pltpu.CompilerParams(use_tc_tiling_on_sc=False),
        out_type=jax.ShapeDtypeStruct((NC, NPAD, DH), jnp.float32),
        scratch_types=[
            pltpu.VMEM((CH2, KW), jnp.int32),
            pltpu.VMEM((CH2, KW), jnp.int32),
            pltpu.VMEM((KW, DH), jnp.float32),
            pltpu.VMEM((KW, DH), jnp.float32),
            pltpu.SemaphoreType.DMA,
            pltpu.SemaphoreType.DMA,
            pltpu.SemaphoreType.DMA,
            pltpu.SemaphoreType.DMA,
            pltpu.VMEM_SHARED((NPAD, DH), jnp.float32),
        ],
    )
    def k(y_hbm, src_hbm, dst_hbm, z_hbm, out_hbm, sbuf, dbuf, rows0, rows1,
          sem0, sem1, sems0, sems1, agg_sp):
        c = lax.axis_index("c")
        s = lax.axis_index("s")
        pltpu.sync_copy(z_hbm, agg_sp.at[pl.ds(s * ROWS_PER_TILE, ROWS_PER_TILE)])
        pltpu.sync_copy(src_hbm.at[s], sbuf)
        pltpu.sync_copy(dst_hbm.at[s], dbuf)
        def shift(i, carry):
            r = i // (KW // 16)
            col = (i % (KW // 16)) * 16
            v = sbuf[r, pl.ds(col, 16)]
            sbuf[r, pl.ds(col, 16)] = v + v + c
            return carry

        lax.fori_loop(0, CH2 * (KW // 16), shift, 0)
        plsc.subcore_barrier()

        pltpu.async_copy(y_hbm.at[sbuf.at[0]], rows0, sem0)
        pltpu.async_copy(y_hbm.at[sbuf.at[1]], rows1, sem1)

        def body(g, carry):
            j0 = 2 * g
            pltpu.make_async_copy(y_hbm.at[sbuf.at[j0]], rows0, sem0).wait()
            pltpu.sync_copy(rows0, agg_sp.at[dbuf.at[j0]], add=True)
            pltpu.async_copy(y_hbm.at[sbuf.at[j0 + 2]], rows0, sem0)
            pltpu.make_async_copy(y_hbm.at[sbuf.at[j0 + 1]], rows1, sem1).wait()
            pltpu.sync_copy(rows1, agg_sp.at[dbuf.at[j0 + 1]], add=True)
            pltpu.async_copy(y_hbm.at[sbuf.at[j0 + 3]], rows1, sem1)
            return carry

        lax.fori_loop(0, CH2 // 2 - 1, body, 0)
        pltpu.make_async_copy(y_hbm.at[sbuf.at[CH2 - 2]], rows0, sem0).wait()
        pltpu.sync_copy(rows0, agg_sp.at[dbuf.at[CH2 - 2]], add=True)
        pltpu.make_async_copy(y_hbm.at[sbuf.at[CH2 - 1]], rows1, sem1).wait()
        pltpu.sync_copy(rows1, agg_sp.at[dbuf.at[CH2 - 1]], add=True)
        plsc.subcore_barrier()
        pltpu.sync_copy(
            agg_sp.at[pl.ds(s * ROWS_PER_TILE, ROWS_PER_TILE)],
            out_hbm.at[c, pl.ds(s * ROWS_PER_TILE, ROWS_PER_TILE)],
        )

    return k(y2, src2, dst2, zeros_rows)


def _tc_scale_matmul(state, Wc, degp):
    BM = 1000

    def body(x_ref, w_ref, d_ref, o_ref, dv_ref):
        deg = d_ref[:, 0:1] + d_ref[:, 1:2] + 1.0
        dv = lax.rsqrt(deg)
        dv_ref[...] = dv
        o_ref[...] = (
            jnp.dot(x_ref[...], w_ref[...], preferred_element_type=jnp.float32)
            * dv
        )

    return pl.pallas_call(
        body,
        grid=(N // BM,),
        in_specs=[
            pl.BlockSpec((BM, D), lambda i: (i, 0)),
            pl.BlockSpec((D, D), lambda i: (0, 0)),
            pl.BlockSpec((BM, NC), lambda i: (i, 0)),
        ],
        out_specs=[
            pl.BlockSpec((BM, D), lambda i: (i, 0)),
            pl.BlockSpec((BM, 1), lambda i: (i, 0)),
        ],
        out_shape=[
            jax.ShapeDtypeStruct((N, D), jnp.float32),
            jax.ShapeDtypeStruct((N, 1), jnp.float32),
        ],
    )(state, Wc, degp)


def _tc_post(aggp, y2, dinv_col, state, af, Amat, bc2, W1a, w1b, b12, W2, b22, W3, b32):
    BM = 2000
    BG = BM // ACT

    def body(agg_ref, y_ref, dv_ref, x_ref, a_ref, A_ref, bc_ref,
             W1_ref, w1b_ref, b1_ref, W2_ref, b2_ref, W3_ref, b3_ref, o_ref):
        aggsum = jnp.concatenate([agg_ref[0], agg_ref[1]], axis=-1)
        conv = jnp.maximum(dv_ref[...] * (aggsum + y_ref[...]) + bc_ref[...], 0.0)
        x = conv + x_ref[...]
        h1 = jnp.maximum(
            jnp.dot(x, W1_ref[...], preferred_element_type=jnp.float32)
            + a_ref[...] * w1b_ref[...]
            + b1_ref[...],
            0.0,
        )
        h2 = jnp.maximum(
            jnp.dot(h1, W2_ref[...], preferred_element_type=jnp.float32)
            + b2_ref[...],
            0.0,
        )
        hs = jnp.dot(A_ref[...], h2, preferred_element_type=jnp.float32)
        o_ref[...] = (
            jnp.dot(hs, W3_ref[...], preferred_element_type=jnp.float32)
            + b3_ref[...]
        )

    return pl.pallas_call(
        body,
        grid=(N // BM,),
        in_specs=[
            pl.BlockSpec((NC, BM, DH), lambda i: (0, i, 0)),
            pl.BlockSpec((BM, D), lambda i: (i, 0)),
            pl.BlockSpec((BM, 1), lambda i: (i, 0)),
            pl.BlockSpec((BM, D), lambda i: (i, 0)),
            pl.BlockSpec((BM, 1), lambda i: (i, 0)),
            pl.BlockSpec((BG, BM), lambda i: (0, 0)),
            pl.BlockSpec((1, D), lambda i: (0, 0)),
            pl.BlockSpec((D, H), lambda i: (0, 0)),
            pl.BlockSpec((1, H), lambda i: (0, 0)),
            pl.BlockSpec((1, H), lambda i: (0, 0)),
            pl.BlockSpec((H, H), lambda i: (0, 0)),
            pl.BlockSpec((1, H), lambda i: (0, 0)),
            pl.BlockSpec((H, 1), lambda i: (0, 0)),
            pl.BlockSpec((1, 1), lambda i: (0, 0)),
        ],
        out_specs=pl.BlockSpec((BG, 1), lambda i: (i, 0)),
        out_shape=jax.ShapeDtypeStruct((N // ACT, 1), jnp.float32),
    )(aggp, y2, dinv_col, state, af, Amat, bc2, W1a, w1b, b12, W2, b22, W3, b32)


def kernel(state, edge_index, action, Wc, bc, W1, b1, W2, b2, W3, b3):
    src2 = edge_index[0].reshape(NS, CH2, KW)
    dst2 = edge_index[1].reshape(NS, CH2, KW)
    dst3 = edge_index[1].reshape(NW, CH, KW)
    af = action.reshape(N, 1)
    zeros_deg = jnp.zeros((DEG_PER_TILE,), jnp.float32)
    ones_kw = jnp.ones((KW,), jnp.float32)
    zeros_rows = jnp.zeros((ROWS_PER_TILE, DH), jnp.float32)
    BM = 2000
    BG = BM // ACT
    Amat = (jnp.arange(BM, dtype=jnp.int32) // ACT
            == jnp.arange(BG, dtype=jnp.int32)[:, None]).astype(jnp.float32)

    degp = _sc_degree(dst3, zeros_deg, ones_kw).reshape(NC, NPAD)
    degp_t = degp.T[:N]
    y, dinv_col = _tc_scale_matmul(state, Wc, degp_t)
    y2flat = y.reshape(NC * N, DH)
    aggp = _sc_aggregate(y2flat, src2, dst2, zeros_rows)
    out2 = _tc_post(
        aggp, y, dinv_col, state, af, Amat,
        bc.reshape(1, D), W1[:D], W1[D:], b1.reshape(1, H),
        W2, b2.reshape(1, H), W3, b3.reshape(1, 1),
    )
    return out2.reshape(N // ACT)
